# baseline (speedup 1.0000x reference)
import functools

import jax
import jax.numpy as jnp
from jax.experimental import pallas as pl
from jax.experimental.pallas import tpu as pltpu

_EPS = 1e-5
_OUT_REAL = 3
_F_PAD = 128
_TILE_B = 16384


def _round_up(x, m):
    return (x + m - 1) // m * m


def _stats_kernel(x_rot_ref, gs_ref, xb_ref, *, batch, tile_b, nt):
    i = pl.program_id(0)
    x = x_rot_ref[...]
    if batch != tile_b * nt:
        row = i * tile_b + jax.lax.broadcasted_iota(jnp.int32, (tile_b, 1), 0)
        x = jnp.where(row < batch, x, 0.0)
    xb = x.astype(jnp.bfloat16)
    ones = jnp.ones((tile_b, 8), jnp.bfloat16)
    if batch != tile_b * nt:
        row = i * tile_b + jax.lax.broadcasted_iota(jnp.int32, (tile_b, 1), 0)
        ones = jnp.where(row < batch, ones, jnp.bfloat16(0.0))
    aug = jnp.concatenate([xb, ones], axis=1)
    g = jax.lax.dot_general(aug, xb, (((0,), (0,)), ((), ())),
                            preferred_element_type=jnp.float32)

    @pl.when(i == 0)
    def _():
        gs_ref[...] = g

    @pl.when(i > 0)
    def _():
        gs_ref[...] += g

    xb_ref[...] = xb


def _main_kernel(xt_ref, xb_ref, gs_ref, w1_ref, b1_ref, gamma_ref, beta_ref,
                 w2_ref, out_ref, w1s_scr, b1s_scr, *, batch, feat):
    i = pl.program_id(0)

    @pl.when(i == 0)
    def _finalize_stats():
        inv_b = jnp.float32(1.0 / batch)
        w1 = w1_ref[...].astype(jnp.float32)
        g = gs_ref[0:_F_PAD, :]
        s = gs_ref[_F_PAD:_F_PAD + 1, :]
        sw = jax.lax.dot_general(s, w1, (((1,), (0,)), ((), ())),
                                 preferred_element_type=jnp.float32)
        gw = jax.lax.dot_general(g, w1, (((0,), (0,)), ((), ())),
                                 preferred_element_type=jnp.float32)
        q = jnp.sum(w1 * gw, axis=0, keepdims=True)
        b1 = b1_ref[...]
        mean = sw * inv_b + b1
        e_sq = q * inv_b + 2.0 * b1 * sw * inv_b + b1 * b1
        var = jnp.maximum(e_sq - mean * mean, 0.0)
        scale = gamma_ref[...] * jax.lax.rsqrt(var + _EPS)
        shift = beta_ref[...] - mean * scale
        w1s_scr[...] = (w1 * scale).astype(jnp.bfloat16)
        b1s_scr[...] = (b1 * scale + shift).reshape(_F_PAD, 1)

    a = jnp.maximum(xt_ref[...].astype(jnp.bfloat16), jnp.bfloat16(0.0))
    fpad = _round_up(feat, 8)
    if fpad != feat:
        a = jnp.pad(a, ((0, fpad - feat), (0, 0)))
    xb = xb_ref[...]
    ht = jax.lax.dot_general(w1s_scr[...], xb, (((0,), (1,)), ((), ())),
                             preferred_element_type=jnp.float32)
    ht = jnp.maximum((ht + b1s_scr[...]).astype(jnp.bfloat16),
                     jnp.bfloat16(0.0))
    cat = jnp.concatenate(
        [a, ht, jnp.ones((8, a.shape[1]), jnp.bfloat16)], axis=0)
    out = jax.lax.dot_general(w2_ref[...], cat, (((0,), (0,)), ((), ())),
                              preferred_element_type=jnp.float32)
    out_ref[...] = out[:_OUT_REAL, :]


def kernel(x_mlp, x_car_rot, w1, b1, gamma, beta, w2a, w2b, b2):
    B = x_mlp.shape[0]
    feat = x_mlp.shape[1]
    xt = jnp.transpose(x_mlp.reshape(B, feat))
    x_rot = x_car_rot.reshape(B, -1)
    R = x_rot.shape[1]

    tile_b = min(_TILE_B, _round_up(B, 128))
    b_pad = _round_up(B, tile_b)
    nt = b_pad // tile_b
    if b_pad != B:
        xt = jnp.pad(xt, ((0, 0), (0, b_pad - B)))
        x_rot = jnp.pad(x_rot, ((0, b_pad - B), (0, 0)))

    w1_b = w1.astype(jnp.bfloat16)
    fpad = _round_up(feat, 8)
    w2_b = jnp.concatenate(
        [w2a[:feat], jnp.zeros((fpad - feat, w2a.shape[1]), jnp.float32),
         w2b, b2, jnp.zeros((7, w2a.shape[1]), jnp.float32)],
        axis=0).astype(jnp.bfloat16)

    gs, xb = pl.pallas_call(
        functools.partial(_stats_kernel, batch=B, tile_b=tile_b, nt=nt),
        grid=(nt,),
        in_specs=[pl.BlockSpec((tile_b, R), lambda i: (i, 0))],
        out_specs=(pl.BlockSpec((_F_PAD + 8, _F_PAD), lambda i: (0, 0)),
                   pl.BlockSpec((tile_b, R), lambda i: (i, 0))),
        out_shape=(jax.ShapeDtypeStruct((_F_PAD + 8, _F_PAD), jnp.float32),
                   jax.ShapeDtypeStruct((b_pad, R), jnp.bfloat16)),
        compiler_params=pltpu.CompilerParams(
            dimension_semantics=("arbitrary",),
            vmem_limit_bytes=40 * (1 << 20),
        ),
    )(x_rot)

    out_t = pl.pallas_call(
        functools.partial(_main_kernel, batch=B, feat=feat),
        grid=(nt,),
        in_specs=[
            pl.BlockSpec((feat, tile_b), lambda i: (0, i)),
            pl.BlockSpec((tile_b, R), lambda i: (i, 0)),
            pl.BlockSpec((_F_PAD + 8, _F_PAD), lambda i: (0, 0)),
            pl.BlockSpec((R, _F_PAD), lambda i: (0, 0)),
            pl.BlockSpec((1, _F_PAD), lambda i: (0, 0)),
            pl.BlockSpec((1, _F_PAD), lambda i: (0, 0)),
            pl.BlockSpec((1, _F_PAD), lambda i: (0, 0)),
            pl.BlockSpec((fpad + R + 8, _F_PAD), lambda i: (0, 0)),
        ],
        out_specs=pl.BlockSpec((_OUT_REAL, tile_b), lambda i: (0, i)),
        out_shape=jax.ShapeDtypeStruct((_OUT_REAL, b_pad), jnp.float32),
        scratch_shapes=[
            pltpu.VMEM((R, _F_PAD), jnp.bfloat16),
            pltpu.VMEM((_F_PAD, 1), jnp.float32),
        ],
        compiler_params=pltpu.CompilerParams(
            dimension_semantics=("arbitrary",),
            vmem_limit_bytes=48 * (1 << 20),
        ),
    )(xt, xb, gs, w1_b, b1, gamma, beta, w2_b)

    out = jnp.transpose(out_t)
    return out[:B] if b_pad != B else out


# speedup vs baseline: 1.6289x; 1.6289x over previous
import functools

import jax
import jax.numpy as jnp
from jax.experimental import pallas as pl
from jax.experimental.pallas import tpu as pltpu

_EPS = 1e-5
_OUT_REAL = 3
_F_PAD = 128
_TILE_B = 16384


def _round_up(x, m):
    return (x + m - 1) // m * m


def _fused_kernel(xt_ref, x_rot_ref, w1_ref, b1_ref, gamma_ref, beta_ref,
                  w2a_ref, w2b_ref, b2_ref, out_ref,
                  x_scr, gs_scr, w1s_scr, b1s_scr, w2_scr,
                  *, batch, tile_b, nt, feat):
    p = pl.program_id(0)
    i = pl.program_id(1)

    @pl.when(p == 0)
    def _stats_phase():
        x = x_rot_ref[...]
        if batch != tile_b * nt:
            row = i * tile_b + jax.lax.broadcasted_iota(jnp.int32, (tile_b, 1), 0)
            x = jnp.where(row < batch, x, 0.0)
        xb = x.astype(jnp.bfloat16)
        ones = jnp.ones((tile_b, 8), jnp.bfloat16)
        if batch != tile_b * nt:
            row = i * tile_b + jax.lax.broadcasted_iota(jnp.int32, (tile_b, 1), 0)
            ones = jnp.where(row < batch, ones, jnp.bfloat16(0.0))
        aug = jnp.concatenate([xb, ones], axis=1)
        g = jax.lax.dot_general(aug, xb, (((0,), (0,)), ((), ())),
                                preferred_element_type=jnp.float32)

        @pl.when(i == 0)
        def _():
            gs_scr[...] = g

        @pl.when(i > 0)
        def _():
            gs_scr[...] += g

        x_scr[pl.ds(i * tile_b, tile_b), :] = xb

    @pl.when((p == 1) & (i == 0))
    def _finalize_stats():
        inv_b = jnp.float32(1.0 / batch)
        w1 = w1_ref[...]
        fpad = _round_up(feat, 8)
        w2_scr[...] = jnp.concatenate(
            [w2a_ref[0:feat, :],
             jnp.zeros((fpad - feat, _F_PAD), jnp.float32),
             w2b_ref[...], b2_ref[...],
             jnp.zeros((7, _F_PAD), jnp.float32)],
            axis=0).astype(jnp.bfloat16)
        g = gs_scr[0:_F_PAD, :]
        s = gs_scr[_F_PAD:_F_PAD + 1, :]
        sw = jax.lax.dot_general(s, w1, (((1,), (0,)), ((), ())),
                                 preferred_element_type=jnp.float32)
        gw = jax.lax.dot_general(g, w1, (((0,), (0,)), ((), ())),
                                 preferred_element_type=jnp.float32)
        q = jnp.sum(w1 * gw, axis=0, keepdims=True)
        b1 = b1_ref[...]
        mean = sw * inv_b + b1
        e_sq = q * inv_b + 2.0 * b1 * sw * inv_b + b1 * b1
        var = jnp.maximum(e_sq - mean * mean, 0.0)
        scale = gamma_ref[...] * jax.lax.rsqrt(var + _EPS)
        shift = beta_ref[...] - mean * scale
        w1s_scr[...] = (w1 * scale).astype(jnp.bfloat16)
        b1s_scr[...] = (b1 * scale + shift).reshape(_F_PAD, 1)

    @pl.when(p == 1)
    def _output_phase():
        a = jnp.maximum(xt_ref[...], jnp.bfloat16(0.0))
        fpad = _round_up(feat, 8)
        if fpad != feat:
            a = jnp.pad(a, ((0, fpad - feat), (0, 0)))
        xb = x_scr[pl.ds(i * tile_b, tile_b), :]
        ht = jax.lax.dot_general(w1s_scr[...], xb, (((0,), (1,)), ((), ())),
                                 preferred_element_type=jnp.float32)
        ht = jnp.maximum((ht + b1s_scr[...]).astype(jnp.bfloat16),
                         jnp.bfloat16(0.0))
        cat = jnp.concatenate(
            [a, ht, jnp.ones((8, a.shape[1]), jnp.bfloat16)], axis=0)
        out = jax.lax.dot_general(w2_scr[...], cat, (((0,), (0,)), ((), ())),
                                  preferred_element_type=jnp.float32)
        out_ref[...] = out[:_OUT_REAL, :]


def kernel(x_mlp, x_car_rot, w1, b1, gamma, beta, w2a, w2b, b2):
    B = x_mlp.shape[0]
    feat = x_mlp.shape[1]
    xt = jnp.transpose(x_mlp.reshape(B, feat)).astype(jnp.bfloat16)
    x_rot = x_car_rot.reshape(B, -1)
    R = x_rot.shape[1]

    tile_b = min(_TILE_B, _round_up(B, 128))
    b_pad = _round_up(B, tile_b)
    nt = b_pad // tile_b
    if b_pad != B:
        xt = jnp.pad(xt, ((0, 0), (0, b_pad - B)))
        x_rot = jnp.pad(x_rot, ((0, b_pad - B), (0, 0)))

    fpad = _round_up(feat, 8)
    vmem_bytes = 50 * (1 << 20)

    out_t = pl.pallas_call(
        functools.partial(_fused_kernel, batch=B, tile_b=tile_b, nt=nt,
                          feat=feat),
        grid=(2, nt),
        in_specs=[
            pl.BlockSpec((feat, tile_b), lambda p, i: (0, p * i)),
            pl.BlockSpec((tile_b, R), lambda p, i: ((1 - p) * i + p * (nt - 1), 0)),
            pl.BlockSpec((R, _F_PAD), lambda p, i: (0, 0)),
            pl.BlockSpec((1, _F_PAD), lambda p, i: (0, 0)),
            pl.BlockSpec((1, _F_PAD), lambda p, i: (0, 0)),
            pl.BlockSpec((1, _F_PAD), lambda p, i: (0, 0)),
            pl.BlockSpec((_F_PAD, _F_PAD), lambda p, i: (0, 0)),
            pl.BlockSpec((_F_PAD, _F_PAD), lambda p, i: (0, 0)),
            pl.BlockSpec((1, _F_PAD), lambda p, i: (0, 0)),
        ],
        out_specs=pl.BlockSpec((_OUT_REAL, tile_b), lambda p, i: (0, p * i)),
        out_shape=jax.ShapeDtypeStruct((_OUT_REAL, b_pad), jnp.float32),
        scratch_shapes=[
            pltpu.VMEM((b_pad, R), jnp.bfloat16),
            pltpu.VMEM((_F_PAD + 8, _F_PAD), jnp.float32),
            pltpu.VMEM((R, _F_PAD), jnp.bfloat16),
            pltpu.VMEM((_F_PAD, 1), jnp.float32),
            pltpu.VMEM((fpad + R + 8, _F_PAD), jnp.bfloat16),
        ],
        compiler_params=pltpu.CompilerParams(
            dimension_semantics=("arbitrary", "arbitrary"),
            vmem_limit_bytes=vmem_bytes,
        ),
    )(xt, x_rot, w1, b1, gamma, beta, w2a, w2b, b2)

    out = jnp.transpose(out_t)
    return out[:B] if b_pad != B else out
